# baseline (speedup 1.0000x reference)
import functools

import jax
import jax.numpy as jnp
from jax import lax
from jax.experimental import pallas as pl
from jax.experimental.pallas import tpu as pltpu
from jax.experimental.pallas import tpu_sc as plsc

B = 16384
NNUM = 13
NCAT = 26
NFEAT = 39
VOCAB = 100000
D = 32
HALF = 16
EPS = 1e-5

NW = 32
ROWS_PER_W = B * NCAT // NW
CHUNK = 128
NCH = ROWS_PER_W // CHUNK
GROUP = 8
NG = NCH // GROUP


def _sc_gather(table_hbm, idx_hbm, out_hbm, idx_v, rows_v, gsem, wsem):
    wid = lax.axis_index("s") * 2 + lax.axis_index("c")
    pltpu.sync_copy(idx_hbm.at[wid], idx_v)
    base = wid * ROWS_PER_W

    def g_desc(c, slot):
        return pltpu.make_async_copy(
            table_hbm.at[idx_v.at[c]], rows_v.at[slot], gsem)

    def w_desc(c, slot):
        return pltpu.make_async_copy(
            rows_v.at[slot], out_hbm.at[pl.ds(base + c * CHUNK, CHUNK)], wsem)

    for j in range(GROUP):
        g_desc(j, j).start()

    def body(g, _):
        bank = lax.rem(g, 2) * GROUP
        nbank = lax.rem(g + 1, 2) * GROUP

        @pl.when(g >= 1)
        def _():
            for j in range(GROUP):
                w_desc((g - 1) * GROUP + j, nbank + j).wait()

        @pl.when(g + 1 < NG)
        def _():
            for j in range(GROUP):
                g_desc((g + 1) * GROUP + j, nbank + j).start()

        for j in range(GROUP):
            g_desc(g * GROUP + j, bank + j).wait()
        for j in range(GROUP):
            w_desc(g * GROUP + j, bank + j).start()
        return 0

    lax.fori_loop(0, NG, body, 0)
    lbank = ((NG - 1) % 2) * GROUP
    for j in range(GROUP):
        w_desc((NG - 1) * GROUP + j, lbank + j).wait()


@functools.cache
def _gather_call():
    return functools.partial(
        pl.kernel,
        out_type=jax.ShapeDtypeStruct((B * NCAT, D), jnp.float32),
        mesh=plsc.VectorSubcoreMesh(core_axis_name="c", subcore_axis_name="s"),
        scratch_types=[
            pltpu.VMEM((NCH, CHUNK), jnp.int32),
            pltpu.VMEM((2 * GROUP, CHUNK, D), jnp.float32),
            pltpu.SemaphoreType.DMA,
            pltpu.SemaphoreType.DMA,
        ],
        compiler_params=pltpu.CompilerParams(use_tc_tiling_on_sc=False),
    )(_sc_gather)


BC = 256


def _tc_body(xn_ref, freq_ref, cat_ref, gam_ref, bet_ref, out_ref):
    gam = gam_ref[0]
    bet = bet_ref[0]
    xn = xn_ref[...]
    freq = freq_ref[...]
    v = xn[:, :, None] * freq[None, :, :]
    s = jnp.sin(v)
    co = jnp.cos(v)
    m = (jnp.sum(s, -1, keepdims=True) + jnp.sum(co, -1, keepdims=True)) / D
    inv = lax.rsqrt(0.5 - m * m + EPS)
    out_ref[:, :NNUM, :HALF] = (s - m) * inv * gam[:HALF] + bet[:HALF]
    out_ref[:, :NNUM, HALF:] = (co - m) * inv * gam[HALF:] + bet[HALF:]

    t = cat_ref[...]
    cm = jnp.mean(t, -1, keepdims=True)
    cvar = jnp.maximum(jnp.mean(t * t, -1, keepdims=True) - cm * cm, 0.0)
    out_ref[:, NNUM:, :] = (t - cm) * lax.rsqrt(cvar + EPS) * gam + bet


_tc_call = pl.pallas_call(
    _tc_body,
    grid=(B // BC,),
    in_specs=[
        pl.BlockSpec((BC, NNUM), lambda i: (i, 0)),
        pl.BlockSpec((NNUM, HALF), lambda i: (0, 0)),
        pl.BlockSpec((BC, NCAT, D), lambda i: (i, 0, 0)),
        pl.BlockSpec((1, D), lambda i: (0, 0)),
        pl.BlockSpec((1, D), lambda i: (0, 0)),
    ],
    out_specs=pl.BlockSpec((BC, NFEAT, D), lambda i: (i, 0, 0)),
    out_shape=jax.ShapeDtypeStruct((B, NFEAT, D), jnp.float32),
    compiler_params=pltpu.CompilerParams(
        vmem_limit_bytes=100 * 1024 * 1024),
)


def kernel(x, numerical_frequencies, emb_tables, ln_gamma, ln_beta):
    xn = x[:, :NNUM]
    flat_idx = (x[:, NNUM:].astype(jnp.int32)
                + jnp.arange(NCAT, dtype=jnp.int32)[None, :] * VOCAB)
    idx = flat_idx.reshape(NW, NCH, CHUNK)
    table = emb_tables.reshape(NCAT * VOCAB, D)
    cat_rows = _gather_call()(table, idx).reshape(B, NCAT, D)
    return _tc_call(xn, numerical_frequencies, cat_rows,
                    ln_gamma.reshape(1, D), ln_beta.reshape(1, D))


# speedup vs baseline: 4.6380x; 4.6380x over previous
import functools

import jax
import jax.numpy as jnp
from jax import lax
from jax.experimental import pallas as pl
from jax.experimental.pallas import tpu as pltpu
from jax.experimental.pallas import tpu_sc as plsc

B = 16384
NNUM = 13
NCAT = 26
NFEAT = 39
VOCAB = 100000
D = 32
HALF = 16
EPS = 1e-5

NW = 32
PLANES = NCAT * D
PPW = PLANES // NW
GCHUNK = 4096
NGCH = B // GCHUNK


def _sc_plane_gather(tab_hbm, idx_hbm, out_hbm, plane_v, idx_v, gout_v):
    wid = lax.axis_index("s") * 2 + lax.axis_index("c")
    p0 = wid * PPW

    def plane_body(p, _):
        f = lax.div(p, D)
        d = lax.rem(p, D)

        @pl.when((p == p0) | (d == 0))
        def _():
            pltpu.sync_copy(idx_hbm.at[f], idx_v)

        pltpu.sync_copy(tab_hbm.at[f, d], plane_v)

        def chunk_body(c, _):
            base = c * GCHUNK

            def gbody(i, _):
                for u in range(8):
                    off = base + i * 128 + u * 16
                    iv = idx_v[pl.ds(off, 16)]
                    gout_v[pl.ds(i * 128 + u * 16, 16)] = plsc.load_gather(
                        plane_v, [iv])
                return 0

            lax.fori_loop(0, GCHUNK // 128, gbody, 0)
            pltpu.sync_copy(gout_v, out_hbm.at[f, d, pl.ds(base, GCHUNK)])
            return 0

        lax.fori_loop(0, NGCH, chunk_body, 0)
        return 0

    lax.fori_loop(p0, p0 + PPW, plane_body, 0)


@functools.cache
def _gather_call():
    return functools.partial(
        pl.kernel,
        out_type=jax.ShapeDtypeStruct((NCAT, D, B), jnp.float32),
        mesh=plsc.VectorSubcoreMesh(core_axis_name="c", subcore_axis_name="s"),
        scratch_types=[
            pltpu.VMEM((VOCAB,), jnp.float32),
            pltpu.VMEM((B,), jnp.int32),
            pltpu.VMEM((GCHUNK,), jnp.float32),
        ],
        compiler_params=pltpu.CompilerParams(needs_layout_passes=False),
    )(_sc_plane_gather)


BCL = 512


def _tc_body(xnT_ref, freq_ref, catT_ref, gam_ref, bet_ref, out_ref):
    gam = gam_ref[...][None, :, :]
    bet = bet_ref[...][None, :, :]
    xnT = xnT_ref[...]
    freq = freq_ref[...]
    v = xnT[:, None, :] * freq[:, :, None]
    s = jnp.sin(v)
    co = jnp.cos(v)
    m = (jnp.sum(s, 1) + jnp.sum(co, 1)) * (1.0 / D)
    inv = lax.rsqrt(0.5 - m * m + EPS)
    sn = (s - m[:, None, :]) * inv[:, None, :]
    cn = (co - m[:, None, :]) * inv[:, None, :]
    out_ref[:NNUM, :HALF] = sn * gam[:, :HALF] + bet[:, :HALF]
    out_ref[:NNUM, HALF:] = cn * gam[:, HALF:] + bet[:, HALF:]

    t = catT_ref[...]
    cm = jnp.mean(t, 1)
    cv = jnp.maximum(jnp.mean(t * t, 1) - cm * cm, 0.0)
    cinv = lax.rsqrt(cv + EPS)
    out_ref[NNUM:] = ((t - cm[:, None, :]) * cinv[:, None, :]) * gam + bet


_tc_call = pl.pallas_call(
    _tc_body,
    grid=(B // BCL,),
    in_specs=[
        pl.BlockSpec((NNUM, BCL), lambda i: (0, i)),
        pl.BlockSpec((NNUM, HALF), lambda i: (0, 0)),
        pl.BlockSpec((NCAT, D, BCL), lambda i: (0, 0, i)),
        pl.BlockSpec((D, 1), lambda i: (0, 0)),
        pl.BlockSpec((D, 1), lambda i: (0, 0)),
    ],
    out_specs=pl.BlockSpec((NFEAT, D, BCL), lambda i: (0, 0, i)),
    out_shape=jax.ShapeDtypeStruct((NFEAT, D, B), jnp.float32),
    compiler_params=pltpu.CompilerParams(
        vmem_limit_bytes=100 * 1024 * 1024),
)


def kernel(x, numerical_frequencies, emb_tables, ln_gamma, ln_beta):
    xT = x.T
    xnT = xT[:NNUM]
    idxT = xT[NNUM:].astype(jnp.int32)
    tabT = emb_tables.transpose(0, 2, 1)
    catT = _gather_call()(tabT, idxT)
    outT = _tc_call(xnT, numerical_frequencies, catT,
                    ln_gamma.reshape(D, 1), ln_beta.reshape(D, 1))
    return outT.transpose(2, 0, 1)


# speedup vs baseline: 4.7124x; 1.0160x over previous
import functools

import jax
import jax.numpy as jnp
from jax import lax
from jax.experimental import pallas as pl
from jax.experimental.pallas import tpu as pltpu
from jax.experimental.pallas import tpu_sc as plsc

B = 16384
NNUM = 13
NCAT = 26
NFEAT = 39
VOCAB = 100000
D = 32
HALF = 16
EPS = 1e-5

NW = 32
PLANES = NCAT * D
PPW = PLANES // NW
GCHUNK = 4096
NGCH = B // GCHUNK


def _sc_plane_gather(tab_hbm, idx_hbm, out_hbm, plane_v, idx_v, gout_v, wsem):
    wid = lax.axis_index("s") * 2 + lax.axis_index("c")
    p0 = wid * PPW
    pltpu.sync_copy(idx_hbm.at[lax.div(p0, D)], idx_v)

    def w_desc(f, d, c, buf):
        return pltpu.make_async_copy(
            gout_v.at[buf],
            out_hbm.at[f, d, pl.ds(c * GCHUNK, GCHUNK)], wsem)

    def plane_body(p, _):
        f = lax.div(p, D)
        d = lax.rem(p, D)

        @pl.when((d == 0) & (p > p0))
        def _():
            pltpu.sync_copy(idx_hbm.at[f], idx_v)

        pltpu.sync_copy(tab_hbm.at[f, d], plane_v)

        def chunk_body(c, _):
            buf = lax.rem(c, 2)
            g = (p - p0) * NGCH + c

            @pl.when(g >= 2)
            def _():
                pltpu.make_async_copy(
                    gout_v.at[buf],
                    out_hbm.at[0, 0, pl.ds(0, GCHUNK)], wsem).wait()

            base = c * GCHUNK

            def gbody(i, _):
                for u in range(16):
                    off = i * 256 + u * 16
                    iv = idx_v[pl.ds(base + off, 16)]
                    gout_v[buf, pl.ds(off, 16)] = plsc.load_gather(
                        plane_v, [iv])
                return 0

            lax.fori_loop(0, GCHUNK // 256, gbody, 0)
            w_desc(f, d, c, buf).start()
            return 0

        lax.fori_loop(0, NGCH, chunk_body, 0)
        return 0

    lax.fori_loop(p0, p0 + PPW, plane_body, 0)
    for buf in range(2):
        pltpu.make_async_copy(
            gout_v.at[buf], out_hbm.at[0, 0, pl.ds(0, GCHUNK)], wsem).wait()


@functools.cache
def _gather_call():
    return functools.partial(
        pl.kernel,
        out_type=jax.ShapeDtypeStruct((NCAT, D, B), jnp.float32),
        mesh=plsc.VectorSubcoreMesh(core_axis_name="c", subcore_axis_name="s"),
        scratch_types=[
            pltpu.VMEM((VOCAB,), jnp.float32),
            pltpu.VMEM((B,), jnp.int32),
            pltpu.VMEM((2, GCHUNK), jnp.float32),
            pltpu.SemaphoreType.DMA,
        ],
        compiler_params=pltpu.CompilerParams(needs_layout_passes=False),
    )(_sc_plane_gather)


BCL = 512


def _tc_body(xnT_ref, freq_ref, catT_ref, gam_ref, bet_ref, out_ref):
    gam = gam_ref[...][None, :, :]
    bet = bet_ref[...][None, :, :]
    xnT = xnT_ref[...]
    freq = freq_ref[...]
    v = xnT[:, None, :] * freq[:, :, None]
    s = jnp.sin(v)
    co = jnp.cos(v)
    m = (jnp.sum(s, 1) + jnp.sum(co, 1)) * (1.0 / D)
    inv = lax.rsqrt(0.5 - m * m + EPS)
    sn = (s - m[:, None, :]) * inv[:, None, :]
    cn = (co - m[:, None, :]) * inv[:, None, :]
    out_ref[:NNUM, :HALF] = sn * gam[:, :HALF] + bet[:, :HALF]
    out_ref[:NNUM, HALF:] = cn * gam[:, HALF:] + bet[:, HALF:]

    t = catT_ref[...]
    cm = jnp.mean(t, 1)
    cv = jnp.maximum(jnp.mean(t * t, 1) - cm * cm, 0.0)
    cinv = lax.rsqrt(cv + EPS)
    out_ref[NNUM:] = ((t - cm[:, None, :]) * cinv[:, None, :]) * gam + bet


_tc_call = pl.pallas_call(
    _tc_body,
    grid=(B // BCL,),
    in_specs=[
        pl.BlockSpec((NNUM, BCL), lambda i: (0, i)),
        pl.BlockSpec((NNUM, HALF), lambda i: (0, 0)),
        pl.BlockSpec((NCAT, D, BCL), lambda i: (0, 0, i)),
        pl.BlockSpec((D, 1), lambda i: (0, 0)),
        pl.BlockSpec((D, 1), lambda i: (0, 0)),
    ],
    out_specs=pl.BlockSpec((NFEAT, D, BCL), lambda i: (0, 0, i)),
    out_shape=jax.ShapeDtypeStruct((NFEAT, D, B), jnp.float32),
    compiler_params=pltpu.CompilerParams(
        vmem_limit_bytes=100 * 1024 * 1024),
)


def kernel(x, numerical_frequencies, emb_tables, ln_gamma, ln_beta):
    xT = x.T
    xnT = xT[:NNUM]
    idxT = xT[NNUM:].astype(jnp.int32)
    tabT = emb_tables.transpose(0, 2, 1)
    catT = _gather_call()(tabT, idxT)
    outT = _tc_call(xnT, numerical_frequencies, catT,
                    ln_gamma.reshape(D, 1), ln_beta.reshape(D, 1))
    return outT.transpose(2, 0, 1)


# speedup vs baseline: 7.4129x; 1.5731x over previous
import functools

import jax
import jax.numpy as jnp
from jax import lax
from jax.experimental import pallas as pl
from jax.experimental.pallas import tpu as pltpu
from jax.experimental.pallas import tpu_sc as plsc

B = 16384
NNUM = 13
NCAT = 26
NFEAT = 39
VOCAB = 100000
D = 32
HALF = 16
EPS = 1e-5

NW = 32
PLANES = NCAT * D
PPW = PLANES // NW
GCHUNK = 4096
NGCH = B // GCHUNK


def _sc_plane_gather(tab_hbm, idx_hbm, out_hbm, plane_v, idx_v, gout_v,
                     wsem, gsem):
    wid = lax.axis_index("s") * 2 + lax.axis_index("c")
    p0 = wid * PPW
    pltpu.sync_copy(idx_hbm.at[lax.div(p0, D)], idx_v)

    def w_desc(f, d, c, buf):
        return pltpu.make_async_copy(
            gout_v.at[buf],
            out_hbm.at[f, d, pl.ds(c * GCHUNK, GCHUNK)], wsem)

    def plane_body(p, _):
        f = lax.div(p, D)
        d = lax.rem(p, D)

        @pl.when((d == 0) & (p > p0))
        def _():
            pltpu.sync_copy(idx_hbm.at[f], idx_v)

        pltpu.sync_copy(tab_hbm.at[f, d], plane_v)

        def chunk_body(c, _):
            buf = lax.rem(c, 2)
            g = (p - p0) * NGCH + c

            @pl.when(g >= 2)
            def _():
                pltpu.make_async_copy(
                    gout_v.at[buf],
                    out_hbm.at[0, 0, pl.ds(0, GCHUNK)], wsem).wait()

            base = c * GCHUNK

            def gbody(i, _):
                ivs = [idx_v[pl.ds(base + i * 256 + u * 16, 16)]
                       for u in range(16)]
                gs = [plsc.load_gather(plane_v, [iv]) for iv in ivs]
                for u in range(16):
                    gout_v[buf, pl.ds(i * 256 + u * 16, 16)] = gs[u]
                return 0

            lax.fori_loop(0, GCHUNK // 256, gbody, 0)
            w_desc(f, d, c, buf).start()
            return 0

        lax.fori_loop(0, NGCH, chunk_body, 0)
        return 0

    lax.fori_loop(p0, p0 + PPW, plane_body, 0)
    for buf in range(2):
        pltpu.make_async_copy(
            gout_v.at[buf], out_hbm.at[0, 0, pl.ds(0, GCHUNK)], wsem).wait()


@functools.cache
def _gather_call():
    return functools.partial(
        pl.kernel,
        out_type=jax.ShapeDtypeStruct((NCAT, D, B), jnp.float32),
        mesh=plsc.VectorSubcoreMesh(core_axis_name="c", subcore_axis_name="s"),
        scratch_types=[
            pltpu.VMEM((VOCAB,), jnp.float32),
            pltpu.VMEM((B,), jnp.int32),
            pltpu.VMEM((2, GCHUNK), jnp.float32),
            pltpu.SemaphoreType.DMA,
            pltpu.SemaphoreType.DMA,
        ],
        compiler_params=pltpu.CompilerParams(needs_layout_passes=False),
    )(_sc_plane_gather)


BCL = 512


def _tc_body(xnT_ref, freq_ref, catT_ref, gam_ref, bet_ref, out_ref):
    gam = gam_ref[...][None, :, :]
    bet = bet_ref[...][None, :, :]
    xnT = xnT_ref[...]
    freq = freq_ref[...]
    v = xnT[:, None, :] * freq[:, :, None]
    s = jnp.sin(v)
    co = jnp.cos(v)
    m = (jnp.sum(s, 1) + jnp.sum(co, 1)) * (1.0 / D)
    inv = lax.rsqrt(0.5 - m * m + EPS)
    sn = (s - m[:, None, :]) * inv[:, None, :]
    cn = (co - m[:, None, :]) * inv[:, None, :]
    out_ref[:NNUM, :HALF] = sn * gam[:, :HALF] + bet[:, :HALF]
    out_ref[:NNUM, HALF:] = cn * gam[:, HALF:] + bet[:, HALF:]

    t = catT_ref[...]
    cm = jnp.mean(t, 1)
    cv = jnp.maximum(jnp.mean(t * t, 1) - cm * cm, 0.0)
    cinv = lax.rsqrt(cv + EPS)
    out_ref[NNUM:] = ((t - cm[:, None, :]) * cinv[:, None, :]) * gam + bet


_tc_call = pl.pallas_call(
    _tc_body,
    grid=(B // BCL,),
    in_specs=[
        pl.BlockSpec((NNUM, BCL), lambda i: (0, i)),
        pl.BlockSpec((NNUM, HALF), lambda i: (0, 0)),
        pl.BlockSpec((NCAT, D, BCL), lambda i: (0, 0, i)),
        pl.BlockSpec((D, 1), lambda i: (0, 0)),
        pl.BlockSpec((D, 1), lambda i: (0, 0)),
    ],
    out_specs=pl.BlockSpec((NFEAT, D, BCL), lambda i: (0, 0, i)),
    out_shape=jax.ShapeDtypeStruct((NFEAT, D, B), jnp.float32),
    compiler_params=pltpu.CompilerParams(
        vmem_limit_bytes=100 * 1024 * 1024),
)


def kernel(x, numerical_frequencies, emb_tables, ln_gamma, ln_beta):
    xT = x.T
    xnT = xT[:NNUM]
    idxT = xT[NNUM:].astype(jnp.int32)
    tabT = emb_tables.transpose(0, 2, 1)
    catT = _gather_call()(tabT, idxT)
    outT = _tc_call(xnT, numerical_frequencies, catT,
                    ln_gamma.reshape(D, 1), ln_beta.reshape(D, 1))
    return outT.transpose(2, 0, 1)


# speedup vs baseline: 8.6741x; 1.1701x over previous
import functools

import jax
import jax.numpy as jnp
from jax import lax
from jax.experimental import pallas as pl
from jax.experimental.pallas import tpu as pltpu
from jax.experimental.pallas import tpu_sc as plsc

B = 16384
NNUM = 13
NCAT = 26
NFEAT = 39
VOCAB = 100000
D = 32
HALF = 16
EPS = 1e-5

NW = 32
NFH = 13
PPW = NFH * D // NW
GCHUNK = 4096
NGCH = B // GCHUNK


def _sc_plane_gather(f_base, tab_hbm, idx_hbm, out_hbm, plane_v, idx_v,
                     gout_v, wsem):
    wid = lax.axis_index("s") * 2 + lax.axis_index("c")
    p0 = f_base * D + wid * PPW
    pltpu.sync_copy(idx_hbm.at[lax.div(p0, D)], idx_v)

    def w_desc(f, d, c, buf):
        return pltpu.make_async_copy(
            gout_v.at[buf],
            out_hbm.at[f - f_base, d, pl.ds(c * GCHUNK, GCHUNK)], wsem)

    def plane_body(p, _):
        f = lax.div(p, D)
        d = lax.rem(p, D)

        @pl.when((d == 0) & (p > p0))
        def _():
            pltpu.sync_copy(idx_hbm.at[f], idx_v)

        pltpu.sync_copy(tab_hbm.at[f, d], plane_v)

        def chunk_body(c, _):
            buf = lax.rem(c, 2)
            g = (p - p0) * NGCH + c

            @pl.when(g >= 2)
            def _():
                pltpu.make_async_copy(
                    gout_v.at[buf],
                    out_hbm.at[0, 0, pl.ds(0, GCHUNK)], wsem).wait()

            base = c * GCHUNK

            def gbody(i, _):
                ivs = [idx_v[pl.ds(base + i * 256 + u * 16, 16)]
                       for u in range(16)]
                gs = [plsc.load_gather(plane_v, [iv]) for iv in ivs]
                for u in range(16):
                    gout_v[buf, pl.ds(i * 256 + u * 16, 16)] = gs[u]
                return 0

            lax.fori_loop(0, GCHUNK // 256, gbody, 0)
            w_desc(f, d, c, buf).start()
            return 0

        lax.fori_loop(0, NGCH, chunk_body, 0)
        return 0

    lax.fori_loop(p0, p0 + PPW, plane_body, 0)
    for buf in range(2):
        pltpu.make_async_copy(
            gout_v.at[buf], out_hbm.at[0, 0, pl.ds(0, GCHUNK)], wsem).wait()


@functools.cache
def _gather_call(f_base):
    return functools.partial(
        pl.kernel,
        out_type=jax.ShapeDtypeStruct((NFH, D, B), jnp.float32),
        mesh=plsc.VectorSubcoreMesh(core_axis_name="c", subcore_axis_name="s"),
        scratch_types=[
            pltpu.VMEM((VOCAB,), jnp.float32),
            pltpu.VMEM((B,), jnp.int32),
            pltpu.VMEM((2, GCHUNK), jnp.float32),
            pltpu.SemaphoreType.DMA,
        ],
        compiler_params=pltpu.CompilerParams(needs_layout_passes=False),
    )(functools.partial(_sc_plane_gather, f_base))


BCL = 512
_TC_PARAMS = pltpu.CompilerParams(vmem_limit_bytes=100 * 1024 * 1024)


def _tc_num_body(xnT_ref, freq_ref, gam_ref, bet_ref, out_ref):
    gam = gam_ref[...][None, :, :]
    bet = bet_ref[...][None, :, :]
    xnT = xnT_ref[...]
    freq = freq_ref[...]
    v = xnT[:, None, :] * freq[:, :, None]
    s = jnp.sin(v)
    co = jnp.cos(v)
    m = (jnp.sum(s, 1) + jnp.sum(co, 1)) * (1.0 / D)
    inv = lax.rsqrt(0.5 - m * m + EPS)
    sn = (s - m[:, None, :]) * inv[:, None, :]
    cn = (co - m[:, None, :]) * inv[:, None, :]
    out_ref[:, :HALF] = sn * gam[:, :HALF] + bet[:, :HALF]
    out_ref[:, HALF:] = cn * gam[:, HALF:] + bet[:, HALF:]


_tc_num = pl.pallas_call(
    _tc_num_body,
    grid=(B // BCL,),
    in_specs=[
        pl.BlockSpec((NNUM, BCL), lambda i: (0, i)),
        pl.BlockSpec((NNUM, HALF), lambda i: (0, 0)),
        pl.BlockSpec((D, 1), lambda i: (0, 0)),
        pl.BlockSpec((D, 1), lambda i: (0, 0)),
    ],
    out_specs=pl.BlockSpec((NNUM, D, BCL), lambda i: (0, 0, i)),
    out_shape=jax.ShapeDtypeStruct((NFEAT, D, B), jnp.float32),
    compiler_params=_TC_PARAMS,
)


def _tc_cat_body(catT_ref, gam_ref, bet_ref, acc_ref, out_ref):
    del acc_ref
    gam = gam_ref[...][None, :, :]
    bet = bet_ref[...][None, :, :]
    t = catT_ref[...]
    cm = jnp.mean(t, 1)
    cv = jnp.maximum(jnp.mean(t * t, 1) - cm * cm, 0.0)
    cinv = lax.rsqrt(cv + EPS)
    out_ref[...] = ((t - cm[:, None, :]) * cinv[:, None, :]) * gam + bet


def _tc_cat(row_block):
    return pl.pallas_call(
        _tc_cat_body,
        grid=(B // BCL,),
        in_specs=[
            pl.BlockSpec((NFH, D, BCL), lambda i: (0, 0, i)),
            pl.BlockSpec((D, 1), lambda i: (0, 0)),
            pl.BlockSpec((D, 1), lambda i: (0, 0)),
            pl.BlockSpec(memory_space=pl.ANY),
        ],
        out_specs=pl.BlockSpec(
            (NFH, D, BCL), lambda i, r=row_block: (r, 0, i)),
        out_shape=jax.ShapeDtypeStruct((NFEAT, D, B), jnp.float32),
        input_output_aliases={3: 0},
        compiler_params=_TC_PARAMS,
    )


def kernel(x, numerical_frequencies, emb_tables, ln_gamma, ln_beta):
    xT = x.T
    xnT = xT[:NNUM]
    idxT = xT[NNUM:].astype(jnp.int32)
    tabT = emb_tables.transpose(0, 2, 1)
    gam = ln_gamma.reshape(D, 1)
    bet = ln_beta.reshape(D, 1)
    cat1 = _gather_call(0)(tabT, idxT)
    cat2 = _gather_call(NFH)(tabT, idxT)
    o = _tc_num(xnT, numerical_frequencies, gam, bet)
    o = _tc_cat(1)(cat1, gam, bet, o)
    o = _tc_cat(2)(cat2, gam, bet, o)
    return o.transpose(2, 0, 1)
